# baseline (speedup 1.0000x reference)
import functools

import jax
import jax.numpy as jnp
from jax import lax
from jax.experimental import pallas as pl
from jax.experimental.pallas import tpu as pltpu
from jax.experimental.pallas import tpu_sc as plsc

F = 26
V = 100001
D = 32
B = 16384

NC = 2
NS = 16
NW = NC * NS

K_CHUNKS = 16
CHUNK = B // K_CHUNKS
N_TASKS = F * K_CHUNKS
TASKS_PER_W = N_TASKS // NW
IDX_ROWS = CHUNK // 128


def kernel(x, tables, bias):
    x2 = x.reshape(F * K_CHUNKS * IDX_ROWS, 128)
    tab = tables.reshape(F * V, D)
    bias2 = bias.reshape(F, D)

    mesh = plsc.VectorSubcoreMesh(
        core_axis_name="c", subcore_axis_name="s",
        num_cores=NC, num_subcores=NS)

    @functools.partial(
        pl.kernel,
        out_type=jax.ShapeDtypeStruct((B, F * D), jnp.float32),
        mesh=mesh,
        compiler_params=pltpu.CompilerParams(use_tc_tiling_on_sc=False),
        scratch_types=[
            pltpu.VMEM((IDX_ROWS, 128), jnp.int32),
            pltpu.VMEM((CHUNK, D), jnp.float32),
            pltpu.VMEM((D,), jnp.float32),
            pltpu.SemaphoreType.DMA,
        ],
    )
    def body(x_ref, tab_ref, bias_ref, out_ref, idx_v, rows_v, bias_v, sem):
        wid = lax.axis_index("s") * NC + lax.axis_index("c")

        def run_task(t, carry):
            task = wid * TASKS_PER_W + t
            f = task // K_CHUNKS
            s = (task % K_CHUNKS) * CHUNK

            pltpu.sync_copy(
                x_ref.at[pl.ds(task * IDX_ROWS, IDX_ROWS)], idx_v)
            pltpu.sync_copy(bias_ref.at[f], bias_v)

            off = f * V
            for i in range(IDX_ROWS):
                for j in range(128 // 16):
                    idx_v[i, pl.ds(j * 16, 16)] = (
                        idx_v[i, pl.ds(j * 16, 16)] + off)

            copies = [
                pltpu.async_copy(tab_ref.at[idx_v.at[i]],
                                 rows_v.at[pl.ds(i * 128, 128)], sem)
                for i in range(IDX_ROWS)
            ]
            for c in copies:
                c.wait()

            b_lo = bias_v[pl.ds(0, 16)]
            b_hi = bias_v[pl.ds(16, 16)]

            def badd(i, c):
                r0 = i * 8
                for k in range(8):
                    rows_v[r0 + k, pl.ds(0, 16)] = (
                        rows_v[r0 + k, pl.ds(0, 16)] + b_lo)
                    rows_v[r0 + k, pl.ds(16, 16)] = (
                        rows_v[r0 + k, pl.ds(16, 16)] + b_hi)
                return c
            lax.fori_loop(0, CHUNK // 8, badd, 0)

            pltpu.sync_copy(rows_v,
                            out_ref.at[pl.ds(s, CHUNK), pl.ds(f * D, D)])
            return carry

        lax.fori_loop(0, TASKS_PER_W, run_task, 0)

    return body(x2, tab, bias2)


# speedup vs baseline: 2.4945x; 2.4945x over previous
import functools

import jax
import jax.numpy as jnp
from jax import lax
from jax.experimental import pallas as pl
from jax.experimental.pallas import tpu as pltpu
from jax.experimental.pallas import tpu_sc as plsc

F = 26
V = 100001
D = 32
B = 16384

NC = 2
NS = 16
NW = NC * NS

BW = B // NW
CH = 256
NCH = BW // CH
NBUF = 4
NT = F * NCH
NSUB = CH // 128


def kernel(x, tables, bias):
    mesh = plsc.VectorSubcoreMesh(
        core_axis_name="c", subcore_axis_name="s",
        num_cores=NC, num_subcores=NS)

    @functools.partial(
        pl.kernel,
        out_type=jax.ShapeDtypeStruct((B, F * D), jnp.float32),
        mesh=mesh,
        compiler_params=pltpu.CompilerParams(use_tc_tiling_on_sc=False),
        scratch_types=[
            pltpu.VMEM((F, BW), jnp.int32),
            pltpu.VMEM((F, D), jnp.float32),
            [pltpu.VMEM((CH, D), jnp.float32) for _ in range(NBUF)],
            [pltpu.SemaphoreType.DMA for _ in range(NBUF)],
            [pltpu.SemaphoreType.DMA for _ in range(NBUF)],
        ],
    )
    def body(x_ref, tab_ref, bias_ref, out_ref, xw_v, bias_v, bufs,
             gsems, osems):
        wid = lax.axis_index("s") * NC + lax.axis_index("c")
        base = wid * BW

        pltpu.sync_copy(x_ref.at[:, pl.ds(base, BW)], xw_v)
        pltpu.sync_copy(bias_ref, bias_v)

        def fire(t, u):
            f, j = divmod(t, NCH)
            for s in range(NSUB):
                pltpu.async_copy(
                    tab_ref.at[f].at[
                        xw_v.at[f, pl.ds(j * CH + s * 128, 128)]],
                    bufs[u].at[pl.ds(s * 128, 128), :], gsems[u])

        def process(t, u):
            f, j = divmod(t, NCH)
            pltpu.make_async_copy(
                out_ref.at[pl.ds(0, CH), pl.ds(0, D)], bufs[u],
                gsems[u]).wait()
            blo = bias_v[f, pl.ds(0, 16)]
            bhi = bias_v[f, pl.ds(16, 16)]
            buf = bufs[u]

            def rows(i, c, buf=buf, blo=blo, bhi=bhi):
                r0 = i * 4
                for k in range(4):
                    buf[r0 + k, pl.ds(0, 16)] = (
                        buf[r0 + k, pl.ds(0, 16)] + blo)
                    buf[r0 + k, pl.ds(16, 16)] = (
                        buf[r0 + k, pl.ds(16, 16)] + bhi)
                return c
            lax.fori_loop(0, CH // 4, rows, 0)
            pltpu.async_copy(
                buf, out_ref.at[pl.ds(base + j * CH, CH), pl.ds(f * D, D)],
                osems[u])

        def drain_write(u):
            pltpu.make_async_copy(
                bufs[u], out_ref.at[pl.ds(0, CH), pl.ds(0, D)],
                osems[u]).wait()

        for t in range(NBUF):
            fire(t, t)
        for t in range(NT):
            u = t % NBUF
            process(t, u)
            nxt = t + NBUF
            if nxt < NT:
                drain_write(u)
                fire(nxt, u)
        for u in range(NBUF):
            drain_write(u)

    return body(x, tables, bias.reshape(F, D))
